# baseline (speedup 1.0000x reference)
import functools

import jax
import jax.numpy as jnp
from jax import lax
from jax.experimental import pallas as pl
from jax.experimental.pallas import tpu as pltpu
from jax.experimental.pallas import tpu_sc as plsc

VOCAB = 1000000
D = 32
C = 100
B = 16384
H = 50

RBLK = 1024
NRB = pl.cdiv(VOCAB, RBLK)
VP = NRB * RBLK

NW = 32
BPW = B // NW
CB = 32
NCHUNK = BPW // CB
IPC = CB * H
GB = 80
NG = IPC // GB


def _tc_repack(vT):
    def body(x_ref, o_ref):
        x = x_ref[...]
        parts = [
            jnp.transpose(lax.slice(x, (0, s * (RBLK // 4)), (D, (s + 1) * (RBLK // 4))))
            for s in range(4)
        ]
        o_ref[...] = jnp.concatenate(parts, axis=1)

    return pl.pallas_call(
        body,
        grid=(NRB,),
        in_specs=[pl.BlockSpec((D, RBLK), lambda i: (0, i))],
        out_specs=pl.BlockSpec((RBLK // 4, 128), lambda i: (i, 0)),
        out_shape=jax.ShapeDtypeStruct((VP // 4, 128), jnp.float32),
    )(vT)


def _sc_gather_mean(words_flat, table):
    info = plsc.get_sparse_core_info()
    nc = info.num_cores
    mesh = plsc.VectorSubcoreMesh(core_axis_name="c", subcore_axis_name="s")

    @functools.partial(
        pl.kernel,
        mesh=mesh,
        out_type=jax.ShapeDtypeStruct((B, D), jnp.float32),
        compiler_params=pltpu.CompilerParams(use_tc_tiling_on_sc=False),
        scratch_types=[
            pltpu.VMEM((IPC,), jnp.int32),
            pltpu.VMEM((IPC, D), jnp.float32),
            pltpu.VMEM((CB, D), jnp.float32),
            pltpu.SemaphoreType.DMA,
        ],
    )
    def k(words_hbm, table_hbm, feat_hbm, idx_v, rows_v, feat_v, sem):
        wid = lax.axis_index("s") * nc + lax.axis_index("c")
        base_elem = wid * BPW

        def chunk_body(ch, carry):
            elem0 = base_elem + ch * CB
            idx_base = pl.multiple_of(elem0 * H, 8)
            pltpu.sync_copy(words_hbm.at[pl.ds(idx_base, IPC)], idx_v)
            copies = [
                pltpu.async_copy(
                    table_hbm.at[idx_v.at[pl.ds(j * GB, GB)]],
                    rows_v.at[pl.ds(j * GB, GB)],
                    sem,
                )
                for j in range(NG)
            ]
            for cp in copies:
                cp.wait()

            def elem_body(e, c2):
                row0 = e * H
                a0 = jnp.zeros((16,), jnp.float32)
                a1 = jnp.zeros((16,), jnp.float32)
                for j in range(H):
                    a0 = a0 + rows_v[row0 + j, pl.ds(0, 16)]
                    a1 = a1 + rows_v[row0 + j, pl.ds(16, 16)]
                feat_v[e, pl.ds(0, 16)] = a0 * (1.0 / H)
                feat_v[e, pl.ds(16, 16)] = a1 * (1.0 / H)
                return c2

            lax.fori_loop(0, CB, elem_body, 0)
            pltpu.sync_copy(feat_v, feat_hbm.at[pl.ds(elem0, CB)])
            return carry

        lax.fori_loop(0, NCHUNK, chunk_body, 0)

    return k(words_flat, table)


def _tc_linear(feat, Wt, b2):
    BB = 2048

    def body(f_ref, w_ref, b_ref, o_ref):
        o_ref[...] = (
            jnp.dot(f_ref[...], w_ref[...], preferred_element_type=jnp.float32)
            + b_ref[...]
        )

    return pl.pallas_call(
        body,
        grid=(B // BB,),
        in_specs=[
            pl.BlockSpec((BB, D), lambda i: (i, 0)),
            pl.BlockSpec((D, C), lambda i: (0, 0)),
            pl.BlockSpec((1, C), lambda i: (0, 0)),
        ],
        out_specs=pl.BlockSpec((BB, C), lambda i: (i, 0)),
        out_shape=jax.ShapeDtypeStruct((B, C), jnp.float32),
    )(feat, Wt, b2)


def kernel(words, vectors, W, b):
    w = words
    wp = (
        jnp.bitwise_and(w, ~1023)
        | jnp.left_shift(jnp.bitwise_and(w, 255), 2)
        | jnp.bitwise_and(jnp.right_shift(w, 8), 3)
    )
    words_flat = wp.reshape(-1)
    table = _tc_repack(vectors.T).reshape(VP, D)
    feat = _sc_gather_mean(words_flat, table)
    return _tc_linear(feat, W.T, b.reshape(1, C))


# speedup vs baseline: 2.0137x; 2.0137x over previous
import functools

import jax
import jax.numpy as jnp
from jax import lax
from jax.experimental import pallas as pl
from jax.experimental.pallas import tpu as pltpu
from jax.experimental.pallas import tpu_sc as plsc

VOCAB = 1000000
D = 32
C = 100
B = 16384
H = 50

RBLK = 8192
NRB = pl.cdiv(VOCAB, RBLK)
VP = NRB * RBLK

NW = 32
BPW = B // NW
CB = 32
NCHUNK = BPW // CB
IPC = CB * H
GB = 80
NG = IPC // GB


def _tc_repack(vT):
    def body(x_ref, o_ref):
        x = x_ref[...]
        parts = [
            jnp.transpose(lax.slice(x, (0, s * (RBLK // 4)), (D, (s + 1) * (RBLK // 4))))
            for s in range(4)
        ]
        o_ref[...] = jnp.concatenate(parts, axis=1)

    return pl.pallas_call(
        body,
        grid=(NRB,),
        in_specs=[pl.BlockSpec((D, RBLK), lambda i: (0, i))],
        out_specs=pl.BlockSpec((RBLK // 4, 128), lambda i: (i, 0)),
        out_shape=jax.ShapeDtypeStruct((VP // 4, 128), jnp.float32),
        compiler_params=pltpu.CompilerParams(
            dimension_semantics=("parallel",),
        ),
    )(vT)


def _sc_gather_mean(words_flat, table):
    info = plsc.get_sparse_core_info()
    nc = info.num_cores
    mesh = plsc.VectorSubcoreMesh(core_axis_name="c", subcore_axis_name="s")

    @functools.partial(
        pl.kernel,
        mesh=mesh,
        out_type=jax.ShapeDtypeStruct((B, D), jnp.float32),
        compiler_params=pltpu.CompilerParams(use_tc_tiling_on_sc=False),
        scratch_types=[
            pltpu.VMEM((IPC,), jnp.int32),
            pltpu.VMEM((IPC, D), jnp.float32),
            pltpu.VMEM((CB, D), jnp.float32),
            pltpu.SemaphoreType.DMA,
        ],
    )
    def k(words_hbm, table_hbm, feat_hbm, idx_v, rows_v, feat_v, sem):
        wid = lax.axis_index("s") * nc + lax.axis_index("c")
        base_elem = wid * BPW

        def chunk_body(ch, carry):
            elem0 = base_elem + ch * CB
            idx_base = pl.multiple_of(elem0 * H, 8)
            pltpu.sync_copy(words_hbm.at[pl.ds(idx_base, IPC)], idx_v)
            copies = [
                pltpu.async_copy(
                    table_hbm.at[idx_v.at[pl.ds(j * GB, GB)]],
                    rows_v.at[pl.ds(j * GB, GB)],
                    sem,
                )
                for j in range(NG)
            ]
            for cp in copies:
                cp.wait()

            def elem_body(e, c2):
                row0 = e * H
                a0 = jnp.zeros((16,), jnp.float32)
                a1 = jnp.zeros((16,), jnp.float32)
                for j in range(H):
                    a0 = a0 + rows_v[row0 + j, pl.ds(0, 16)]
                    a1 = a1 + rows_v[row0 + j, pl.ds(16, 16)]
                feat_v[e, pl.ds(0, 16)] = a0 * (1.0 / H)
                feat_v[e, pl.ds(16, 16)] = a1 * (1.0 / H)
                return c2

            lax.fori_loop(0, CB, elem_body, 0)
            pltpu.sync_copy(feat_v, feat_hbm.at[pl.ds(elem0, CB)])
            return carry

        lax.fori_loop(0, NCHUNK, chunk_body, 0)

    return k(words_flat, table)


def _tc_linear(feat, Wt, b2):
    BB = 2048

    def body(f_ref, w_ref, b_ref, o_ref):
        o_ref[...] = (
            jnp.dot(f_ref[...], w_ref[...], preferred_element_type=jnp.float32)
            + b_ref[...]
        )

    return pl.pallas_call(
        body,
        grid=(B // BB,),
        in_specs=[
            pl.BlockSpec((BB, D), lambda i: (i, 0)),
            pl.BlockSpec((D, C), lambda i: (0, 0)),
            pl.BlockSpec((1, C), lambda i: (0, 0)),
        ],
        out_specs=pl.BlockSpec((BB, C), lambda i: (i, 0)),
        out_shape=jax.ShapeDtypeStruct((B, C), jnp.float32),
    )(feat, Wt, b2)


def kernel(words, vectors, W, b):
    w = words
    q = RBLK // 4
    qbits = q.bit_length() - 1
    wp = (
        jnp.bitwise_and(w, ~(RBLK - 1))
        | jnp.left_shift(jnp.bitwise_and(w, q - 1), 2)
        | jnp.bitwise_and(jnp.right_shift(w, qbits), 3)
    )
    words_flat = wp.reshape(-1)
    table = _tc_repack(vectors.T).reshape(VP, D)
    feat = _sc_gather_mean(words_flat, table)
    return _tc_linear(feat, W.T, b.reshape(1, C))


# speedup vs baseline: 2.0924x; 1.0391x over previous
import functools

import jax
import jax.numpy as jnp
from jax import lax
from jax.experimental import pallas as pl
from jax.experimental.pallas import tpu as pltpu
from jax.experimental.pallas import tpu_sc as plsc

VOCAB = 1000000
D = 32
C = 100
B = 16384
H = 50

RBLK = 8192
NRB = pl.cdiv(VOCAB, RBLK)
VP = NRB * RBLK

NW = 32
BPW = B // NW
CB = 32
NCHUNK = BPW // CB
IPC = CB * H
GB = 80
NG = IPC // GB


def _tc_repack(vT):
    def body(x_ref, o_ref):
        x = x_ref[...]
        parts = [
            jnp.transpose(lax.slice(x, (0, s * (RBLK // 4)), (D, (s + 1) * (RBLK // 4))))
            for s in range(4)
        ]
        o_ref[...] = jnp.concatenate(parts, axis=1)

    return pl.pallas_call(
        body,
        grid=(NRB,),
        in_specs=[pl.BlockSpec((D, RBLK), lambda i: (0, i))],
        out_specs=pl.BlockSpec((RBLK // 4, 128), lambda i: (i, 0)),
        out_shape=jax.ShapeDtypeStruct((VP // 4, 128), jnp.float32),
        compiler_params=pltpu.CompilerParams(
            dimension_semantics=("parallel",),
        ),
    )(vT)


def _sc_gather_mean(words_flat, table):
    info = plsc.get_sparse_core_info()
    nc = info.num_cores
    mesh = plsc.VectorSubcoreMesh(core_axis_name="c", subcore_axis_name="s")

    @functools.partial(
        pl.kernel,
        mesh=mesh,
        out_type=jax.ShapeDtypeStruct((B, 128), jnp.float32),
        compiler_params=pltpu.CompilerParams(use_tc_tiling_on_sc=False),
        scratch_types=[
            pltpu.VMEM((IPC,), jnp.int32),
            pltpu.VMEM((IPC, D), jnp.float32),
            pltpu.VMEM((CB, 128), jnp.float32),
            pltpu.SemaphoreType.DMA,
        ],
    )
    def k(words_hbm, table_hbm, feat_hbm, idx_v, rows_v, feat_v, sem):
        wid = lax.axis_index("s") * nc + lax.axis_index("c")
        base_elem = wid * BPW

        def zero_body(e, c0):
            for g in range(2, 8):
                feat_v[e, pl.ds(16 * g, 16)] = jnp.zeros((16,), jnp.float32)
            return c0

        lax.fori_loop(0, CB, zero_body, 0)

        def chunk_body(ch, carry):
            elem0 = base_elem + ch * CB
            idx_base = pl.multiple_of(elem0 * H, 8)
            pltpu.sync_copy(words_hbm.at[pl.ds(idx_base, IPC)], idx_v)
            copies = [
                pltpu.async_copy(
                    table_hbm.at[idx_v.at[pl.ds(j * GB, GB)]],
                    rows_v.at[pl.ds(j * GB, GB)],
                    sem,
                )
                for j in range(NG)
            ]
            for cp in copies:
                cp.wait()

            def elem_body(e, c2):
                row0 = e * H
                a0 = jnp.zeros((16,), jnp.float32)
                a1 = jnp.zeros((16,), jnp.float32)
                for j in range(H):
                    a0 = a0 + rows_v[row0 + j, pl.ds(0, 16)]
                    a1 = a1 + rows_v[row0 + j, pl.ds(16, 16)]
                feat_v[e, pl.ds(0, 16)] = a0 * (1.0 / H)
                feat_v[e, pl.ds(16, 16)] = a1 * (1.0 / H)
                return c2

            lax.fori_loop(0, CB, elem_body, 0)
            pltpu.sync_copy(feat_v, feat_hbm.at[pl.ds(elem0, CB)])
            return carry

        lax.fori_loop(0, NCHUNK, chunk_body, 0)

    return k(words_flat, table)


def _tc_linear(feat, W2, b2):
    BB = 2048

    def body(w_ref, f_ref, b_ref, o_ref):
        o_ref[...] = (
            lax.dot_general(
                w_ref[...],
                f_ref[...],
                (((1,), (1,)), ((), ())),
                preferred_element_type=jnp.float32,
            )
            + b_ref[...]
        )

    return pl.pallas_call(
        body,
        grid=(B // BB,),
        in_specs=[
            pl.BlockSpec((C, 128), lambda i: (0, 0)),
            pl.BlockSpec((BB, 128), lambda i: (i, 0)),
            pl.BlockSpec((C, 1), lambda i: (0, 0)),
        ],
        out_specs=pl.BlockSpec((C, BB), lambda i: (0, i)),
        out_shape=jax.ShapeDtypeStruct((C, B), jnp.float32),
        compiler_params=pltpu.CompilerParams(
            dimension_semantics=("parallel",),
        ),
    )(W2, feat, b2)


def kernel(words, vectors, W, b):
    w = words
    q = RBLK // 4
    qbits = q.bit_length() - 1
    wp = (
        jnp.bitwise_and(w, ~(RBLK - 1))
        | jnp.left_shift(jnp.bitwise_and(w, q - 1), 2)
        | jnp.bitwise_and(jnp.right_shift(w, qbits), 3)
    )
    words_flat = wp.reshape(-1)
    table = _tc_repack(vectors.T).reshape(VP, D)
    feat = _sc_gather_mean(words_flat, table)
    W2 = jnp.pad(W, ((0, 0), (0, 128 - D)))
    out_t = _tc_linear(feat, W2, b.reshape(C, 1))
    return out_t.T


# speedup vs baseline: 2.1049x; 1.0060x over previous
import functools

import jax
import jax.numpy as jnp
from jax import lax
from jax.experimental import pallas as pl
from jax.experimental.pallas import tpu as pltpu
from jax.experimental.pallas import tpu_sc as plsc

VOCAB = 1000000
D = 32
C = 100
B = 16384
H = 50

RBLK = 16384
NRB = pl.cdiv(VOCAB, RBLK)
VP = NRB * RBLK

NW = 32
BPW = B // NW
CB = 32
NCHUNK = BPW // CB
IPC = CB * H
GB = 80
NG = IPC // GB


def _tc_repack(vT):
    def body(x_ref, o_ref):
        x = x_ref[...]
        for s in range(4):
            o_ref[:, 32 * s:32 * s + 32] = jnp.transpose(
                lax.slice(x, (0, s * (RBLK // 4)), (D, (s + 1) * (RBLK // 4)))
            )

    return pl.pallas_call(
        body,
        grid=(NRB,),
        in_specs=[pl.BlockSpec((D, RBLK), lambda i: (0, i))],
        out_specs=pl.BlockSpec((RBLK // 4, 128), lambda i: (i, 0)),
        out_shape=jax.ShapeDtypeStruct((VP // 4, 128), jnp.float32),
        compiler_params=pltpu.CompilerParams(
            dimension_semantics=("parallel",),
        ),
    )(vT)


def _sc_gather_mean(words_flat, table):
    info = plsc.get_sparse_core_info()
    nc = info.num_cores
    mesh = plsc.VectorSubcoreMesh(core_axis_name="c", subcore_axis_name="s")

    @functools.partial(
        pl.kernel,
        mesh=mesh,
        out_type=jax.ShapeDtypeStruct((B, 128), jnp.float32),
        compiler_params=pltpu.CompilerParams(use_tc_tiling_on_sc=False),
        scratch_types=[
            pltpu.VMEM((IPC,), jnp.int32),
            pltpu.VMEM((IPC, D), jnp.float32),
            pltpu.VMEM((CB, 128), jnp.float32),
            pltpu.SemaphoreType.DMA,
        ],
    )
    def k(words_hbm, table_hbm, feat_hbm, idx_v, rows_v, feat_v, sem):
        wid = lax.axis_index("s") * nc + lax.axis_index("c")
        base_elem = wid * BPW

        def zero_body(e, c0):
            for g in range(2, 8):
                feat_v[e, pl.ds(16 * g, 16)] = jnp.zeros((16,), jnp.float32)
            return c0

        lax.fori_loop(0, CB, zero_body, 0)

        def chunk_body(ch, carry):
            elem0 = base_elem + ch * CB
            idx_base = pl.multiple_of(elem0 * H, 8)
            pltpu.sync_copy(words_hbm.at[pl.ds(idx_base, IPC)], idx_v)
            copies = [
                pltpu.async_copy(
                    table_hbm.at[idx_v.at[pl.ds(j * GB, GB)]],
                    rows_v.at[pl.ds(j * GB, GB)],
                    sem,
                )
                for j in range(NG)
            ]
            for cp in copies:
                cp.wait()

            def elem_body(e, c2):
                row0 = e * H
                a0 = jnp.zeros((16,), jnp.float32)
                a1 = jnp.zeros((16,), jnp.float32)
                for j in range(H):
                    a0 = a0 + rows_v[row0 + j, pl.ds(0, 16)]
                    a1 = a1 + rows_v[row0 + j, pl.ds(16, 16)]
                feat_v[e, pl.ds(0, 16)] = a0 * (1.0 / H)
                feat_v[e, pl.ds(16, 16)] = a1 * (1.0 / H)
                return c2

            lax.fori_loop(0, CB, elem_body, 0)
            pltpu.sync_copy(feat_v, feat_hbm.at[pl.ds(elem0, CB)])
            return carry

        lax.fori_loop(0, NCHUNK, chunk_body, 0)

    return k(words_flat, table)


def _tc_linear(feat, W2, b2):
    BB = 2048

    def body(w_ref, f_ref, b_ref, o_ref):
        o_ref[...] = (
            lax.dot_general(
                w_ref[...],
                f_ref[...],
                (((1,), (1,)), ((), ())),
                preferred_element_type=jnp.float32,
            )
            + b_ref[...]
        )

    return pl.pallas_call(
        body,
        grid=(B // BB,),
        in_specs=[
            pl.BlockSpec((C, 128), lambda i: (0, 0)),
            pl.BlockSpec((BB, 128), lambda i: (i, 0)),
            pl.BlockSpec((C, 1), lambda i: (0, 0)),
        ],
        out_specs=pl.BlockSpec((C, BB), lambda i: (0, i)),
        out_shape=jax.ShapeDtypeStruct((C, B), jnp.float32),
        compiler_params=pltpu.CompilerParams(
            dimension_semantics=("parallel",),
        ),
    )(W2, feat, b2)


def kernel(words, vectors, W, b):
    w = words
    q = RBLK // 4
    qbits = q.bit_length() - 1
    wp = (
        jnp.bitwise_and(w, ~(RBLK - 1))
        | jnp.left_shift(jnp.bitwise_and(w, q - 1), 2)
        | jnp.bitwise_and(jnp.right_shift(w, qbits), 3)
    )
    words_flat = wp.reshape(-1)
    table = _tc_repack(vectors.T).reshape(VP, D)
    feat = _sc_gather_mean(words_flat, table)
    W2 = jnp.pad(W, ((0, 0), (0, 128 - D)))
    out_t = _tc_linear(feat, W2, b.reshape(C, 1))
    return out_t.T
